# baseline (speedup 1.0000x reference)
import functools

import jax
import jax.numpy as jnp
from jax import lax
from jax.experimental import pallas as pl
from jax.experimental.pallas import tpu as pltpu
from jax.experimental.pallas import tpu_sc as plsc

D_MODEL = 64
SCALE = 8.0
NC = 2
NS = 16
NW = NC * NS
SEQ = 200
SPLITS = ((0, 104), (104, 96))
NBUF = 4
LEAD = 2
FMT_BLOCK = 2048


def _fmt_body(t_ref, o_ref):
    t8 = (t_ref[...] * SCALE).T
    o_ref[...] = jnp.concatenate([t8, t8], axis=1)


def _format_table(table_t):
    vocab = table_t.shape[1]
    return pl.pallas_call(
        _fmt_body,
        grid=(pl.cdiv(vocab, FMT_BLOCK),),
        in_specs=[pl.BlockSpec((D_MODEL, FMT_BLOCK), lambda i: (0, i))],
        out_specs=pl.BlockSpec((FMT_BLOCK, 2 * D_MODEL), lambda i: (i, 0)),
        out_shape=jax.ShapeDtypeStruct((vocab, 2 * D_MODEL), jnp.float32),
        compiler_params=pltpu.CompilerParams(
            dimension_semantics=("parallel",)),
    )(table_t)


def _gather_body(table_hbm, x_hbm, out_hbm, idx_v, rows, gsem, osem, isem):
    rows_per_w = x_hbm.shape[0] // NW
    wid = lax.axis_index("s") * NC + lax.axis_index("c")
    rbase = wid * rows_per_w

    def idx_start(slot, row):
        pltpu.make_async_copy(
            x_hbm.at[rbase + row], idx_v.at[slot], isem.at[slot]
        ).start()

    def idx_wait(slot):
        pltpu.make_async_copy(
            x_hbm.at[rbase], idx_v.at[slot], isem.at[slot]
        ).wait()

    def gather_start(b, slot):
        for off, n in SPLITS:
            pltpu.make_async_copy(
                table_hbm.at[idx_v.at[slot].at[pl.ds(off, n)]],
                rows.at[b].at[pl.ds(off, n)],
                gsem.at[b],
            ).start()

    def gather_wait(b):
        for off, n in SPLITS:
            pltpu.make_async_copy(
                table_hbm.at[idx_v.at[0].at[pl.ds(off, n)]],
                rows.at[b].at[pl.ds(off, n)],
                gsem.at[b],
            ).wait()

    def out_start(b, row):
        pltpu.make_async_copy(
            rows.at[b],
            out_hbm.at[pl.ds((rbase + row) * SEQ, SEQ)],
            osem.at[b],
        ).start()

    def out_wait(b):
        pltpu.make_async_copy(
            rows.at[b], out_hbm.at[pl.ds(0, SEQ)], osem.at[b]
        ).wait()

    for j in range(NBUF):
        idx_start(j, j)
    for j in range(LEAD):
        idx_wait(j)
        gather_start(j, j)

    for i in range(NBUF):
        b, s = i, (i + LEAD) % NBUF
        gather_wait(b)
        out_start(b, i)
        idx_start(b, i + NBUF)
        if i >= LEAD:
            out_wait(s)
        idx_wait(s)
        gather_start(s, s)

    @pl.loop(1, rows_per_w // NBUF)
    def _group(g):
        for b in range(NBUF):
            row = g * NBUF + b
            s = (b + LEAD) % NBUF
            gather_wait(b)
            out_start(b, row)

            @pl.when(row + NBUF < rows_per_w)
            def _():
                idx_start(b, row + NBUF)

            @pl.when(row + LEAD < rows_per_w)
            def _():
                out_wait(s)
                idx_wait(s)
                gather_start(s, s)

    for b in range(NBUF):
        out_wait(b)


def _sc_gather(table_f, x):
    n_tok = x.shape[0] * x.shape[1]
    mesh = plsc.VectorSubcoreMesh(core_axis_name="c", subcore_axis_name="s")
    run = pl.kernel(
        _gather_body,
        out_type=jax.ShapeDtypeStruct((n_tok, 2 * D_MODEL), jnp.float32),
        mesh=mesh,
        compiler_params=pltpu.CompilerParams(use_tc_tiling_on_sc=False),
        scratch_types=[
            pltpu.VMEM((NBUF, SEQ), jnp.int32),
            pltpu.VMEM((NBUF, SEQ, 2 * D_MODEL), jnp.float32),
            pltpu.SemaphoreType.DMA((NBUF,)),
            pltpu.SemaphoreType.DMA((NBUF,)),
            pltpu.SemaphoreType.DMA((NBUF,)),
        ],
    )
    return run(table_f, x)


def kernel(x, table):
    table_f = _format_table(table.T)
    out = _sc_gather(table_f, x)
    return out[:, :D_MODEL].reshape(x.shape[0], x.shape[1], D_MODEL)


# speedup vs baseline: 1.2406x; 1.2406x over previous
import functools

import jax
import jax.numpy as jnp
from jax import lax
from jax.experimental import pallas as pl
from jax.experimental.pallas import tpu as pltpu
from jax.experimental.pallas import tpu_sc as plsc

D_MODEL = 64
SCALE = 8.0
NC = 2
NS = 16
NW = NC * NS
SEQ = 200
SPLITS = ((0, 104), (104, 96))
NBUF = 4
LEAD = 2
FMT_BLOCK = 8192


def _fmt_body(t_ref, o_ref):
    t8 = (t_ref[...] * SCALE).T
    o_ref[...] = jnp.concatenate([t8, t8], axis=1)


def _format_table(table_t):
    vocab = table_t.shape[1]
    return pl.pallas_call(
        _fmt_body,
        grid=(pl.cdiv(vocab, FMT_BLOCK),),
        in_specs=[pl.BlockSpec((D_MODEL, FMT_BLOCK), lambda i: (0, i))],
        out_specs=pl.BlockSpec((FMT_BLOCK, 2 * D_MODEL), lambda i: (i, 0)),
        out_shape=jax.ShapeDtypeStruct((vocab, 2 * D_MODEL), jnp.float32),
        compiler_params=pltpu.CompilerParams(
            dimension_semantics=("arbitrary",)),
    )(table_t)


def _gather_body(table_hbm, x_hbm, out_hbm, idx_v, rows, gsem, osem, isem):
    rows_per_w = x_hbm.shape[0] // NW
    wid = lax.axis_index("s") * NC + lax.axis_index("c")
    rbase = wid * rows_per_w

    def idx_start(slot, row):
        pltpu.make_async_copy(
            x_hbm.at[rbase + row], idx_v.at[slot], isem.at[slot]
        ).start()

    def idx_wait(slot):
        pltpu.make_async_copy(
            x_hbm.at[rbase], idx_v.at[slot], isem.at[slot]
        ).wait()

    def gather_start(b, slot):
        for off, n in SPLITS:
            pltpu.make_async_copy(
                table_hbm.at[idx_v.at[slot].at[pl.ds(off, n)]],
                rows.at[b].at[pl.ds(off, n)],
                gsem.at[b],
            ).start()

    def gather_wait(b):
        for off, n in SPLITS:
            pltpu.make_async_copy(
                table_hbm.at[idx_v.at[0].at[pl.ds(off, n)]],
                rows.at[b].at[pl.ds(off, n)],
                gsem.at[b],
            ).wait()

    def out_start(b, row):
        pltpu.make_async_copy(
            rows.at[b],
            out_hbm.at[pl.ds((rbase + row) * SEQ, SEQ)],
            osem.at[b],
        ).start()

    def out_wait(b):
        pltpu.make_async_copy(
            rows.at[b], out_hbm.at[pl.ds(0, SEQ)], osem.at[b]
        ).wait()

    for j in range(NBUF):
        idx_start(j, j)
    for j in range(LEAD):
        idx_wait(j)
        gather_start(j, j)

    for i in range(NBUF):
        b, s = i, (i + LEAD) % NBUF
        gather_wait(b)
        out_start(b, i)
        idx_start(b, i + NBUF)
        if i >= LEAD:
            out_wait(s)
        idx_wait(s)
        gather_start(s, s)

    @pl.loop(1, rows_per_w // NBUF)
    def _group(g):
        for b in range(NBUF):
            row = g * NBUF + b
            s = (b + LEAD) % NBUF
            gather_wait(b)
            out_start(b, row)

            @pl.when(row + NBUF < rows_per_w)
            def _():
                idx_start(b, row + NBUF)

            @pl.when(row + LEAD < rows_per_w)
            def _():
                out_wait(s)
                idx_wait(s)
                gather_start(s, s)

    for b in range(NBUF):
        out_wait(b)


def _sc_gather(table_f, x):
    n_tok = x.shape[0] * x.shape[1]
    mesh = plsc.VectorSubcoreMesh(core_axis_name="c", subcore_axis_name="s")
    run = pl.kernel(
        _gather_body,
        out_type=jax.ShapeDtypeStruct((n_tok, 2 * D_MODEL), jnp.float32),
        mesh=mesh,
        compiler_params=pltpu.CompilerParams(use_tc_tiling_on_sc=False),
        scratch_types=[
            pltpu.VMEM((NBUF, SEQ), jnp.int32),
            pltpu.VMEM((NBUF, SEQ, 2 * D_MODEL), jnp.float32),
            pltpu.SemaphoreType.DMA((NBUF,)),
            pltpu.SemaphoreType.DMA((NBUF,)),
            pltpu.SemaphoreType.DMA((NBUF,)),
        ],
    )
    return run(table_f, x)


def kernel(x, table):
    table_f = _format_table(table.T)
    out = _sc_gather(table_f, x)
    return out[:, :D_MODEL].reshape(x.shape[0], x.shape[1], D_MODEL)


# speedup vs baseline: 1.2914x; 1.0410x over previous
import functools

import jax
import jax.numpy as jnp
from jax import lax
from jax.experimental import pallas as pl
from jax.experimental.pallas import tpu as pltpu
from jax.experimental.pallas import tpu_sc as plsc

D_MODEL = 64
SCALE = 8.0
NC = 2
NS = 16
NW = NC * NS
SEQ = 200
SPLITS = ((0, 104), (104, 96))
NBUF = 4
LEAD = 2
FMT_BLOCK = 16384


def _fmt_body(t_ref, o_ref):
    t8 = (t_ref[...] * SCALE).T
    o_ref[...] = jnp.concatenate([t8, t8], axis=1)


def _format_table(table_t):
    vocab = table_t.shape[1]
    return pl.pallas_call(
        _fmt_body,
        grid=(pl.cdiv(vocab, FMT_BLOCK),),
        in_specs=[pl.BlockSpec((D_MODEL, FMT_BLOCK), lambda i: (0, i))],
        out_specs=pl.BlockSpec((FMT_BLOCK, 2 * D_MODEL), lambda i: (i, 0)),
        out_shape=jax.ShapeDtypeStruct((vocab, 2 * D_MODEL), jnp.float32),
        compiler_params=pltpu.CompilerParams(
            dimension_semantics=("arbitrary",)),
    )(table_t)


def _gather_body(table_hbm, x_hbm, out_hbm, idx_v, rows, gsem, osem, isem):
    rows_per_w = x_hbm.shape[0] // NW
    wid = lax.axis_index("s") * NC + lax.axis_index("c")
    rbase = wid * rows_per_w

    def idx_start(slot, row):
        pltpu.make_async_copy(
            x_hbm.at[rbase + row], idx_v.at[slot], isem.at[slot]
        ).start()

    def idx_wait(slot):
        pltpu.make_async_copy(
            x_hbm.at[rbase], idx_v.at[slot], isem.at[slot]
        ).wait()

    def gather_start(b, slot):
        for off, n in SPLITS:
            pltpu.make_async_copy(
                table_hbm.at[idx_v.at[slot].at[pl.ds(off, n)]],
                rows.at[b].at[pl.ds(off, n)],
                gsem.at[b],
            ).start()

    def gather_wait(b):
        for off, n in SPLITS:
            pltpu.make_async_copy(
                table_hbm.at[idx_v.at[0].at[pl.ds(off, n)]],
                rows.at[b].at[pl.ds(off, n)],
                gsem.at[b],
            ).wait()

    def out_start(b, row):
        pltpu.make_async_copy(
            rows.at[b],
            out_hbm.at[pl.ds((rbase + row) * SEQ, SEQ)],
            osem.at[b],
        ).start()

    def out_wait(b):
        pltpu.make_async_copy(
            rows.at[b], out_hbm.at[pl.ds(0, SEQ)], osem.at[b]
        ).wait()

    for j in range(NBUF):
        idx_start(j, j)
    for j in range(LEAD):
        idx_wait(j)
        gather_start(j, j)

    for i in range(NBUF):
        b, s = i, (i + LEAD) % NBUF
        gather_wait(b)
        out_start(b, i)
        idx_start(b, i + NBUF)
        if i >= LEAD:
            out_wait(s)
        idx_wait(s)
        gather_start(s, s)

    @pl.loop(1, rows_per_w // NBUF)
    def _group(g):
        for b in range(NBUF):
            row = g * NBUF + b
            s = (b + LEAD) % NBUF
            gather_wait(b)
            out_start(b, row)

            @pl.when(row + NBUF < rows_per_w)
            def _():
                idx_start(b, row + NBUF)

            @pl.when(row + LEAD < rows_per_w)
            def _():
                out_wait(s)
                idx_wait(s)
                gather_start(s, s)

    for b in range(NBUF):
        out_wait(b)


def _sc_gather(table_f, x):
    n_tok = x.shape[0] * x.shape[1]
    mesh = plsc.VectorSubcoreMesh(core_axis_name="c", subcore_axis_name="s")
    run = pl.kernel(
        _gather_body,
        out_type=jax.ShapeDtypeStruct((n_tok, 2 * D_MODEL), jnp.float32),
        mesh=mesh,
        compiler_params=pltpu.CompilerParams(use_tc_tiling_on_sc=False),
        scratch_types=[
            pltpu.VMEM((NBUF, SEQ), jnp.int32),
            pltpu.VMEM((NBUF, SEQ, 2 * D_MODEL), jnp.float32),
            pltpu.SemaphoreType.DMA((NBUF,)),
            pltpu.SemaphoreType.DMA((NBUF,)),
            pltpu.SemaphoreType.DMA((NBUF,)),
        ],
    )
    return run(table_f, x)


def kernel(x, table):
    table_f = _format_table(table.T)
    out = _sc_gather(table_f, x)
    return out[:, :D_MODEL].reshape(x.shape[0], x.shape[1], D_MODEL)


# speedup vs baseline: 1.3127x; 1.0165x over previous
import functools

import jax
import jax.numpy as jnp
from jax import lax
from jax.experimental import pallas as pl
from jax.experimental.pallas import tpu as pltpu
from jax.experimental.pallas import tpu_sc as plsc

D_MODEL = 64
SCALE = 8.0
NC = 2
NS = 16
NW = NC * NS
SEQ = 200
SPLITS = ((0, 104), (104, 96))
NBUF = 4
LEAD = 2
FMT_BLOCK = 24576


def _fmt_body(t_ref, o_ref):
    t8 = (t_ref[...] * SCALE).T
    o_ref[...] = jnp.concatenate([t8, t8], axis=1)


def _format_table(table_t):
    vocab = table_t.shape[1]
    return pl.pallas_call(
        _fmt_body,
        grid=(pl.cdiv(vocab, FMT_BLOCK),),
        in_specs=[pl.BlockSpec((D_MODEL, FMT_BLOCK), lambda i: (0, i))],
        out_specs=pl.BlockSpec((FMT_BLOCK, 2 * D_MODEL), lambda i: (i, 0)),
        out_shape=jax.ShapeDtypeStruct((vocab, 2 * D_MODEL), jnp.float32),
        compiler_params=pltpu.CompilerParams(
            dimension_semantics=("arbitrary",)),
    )(table_t)


def _gather_body(table_hbm, x_hbm, out_hbm, idx_v, rows, gsem, osem, isem):
    rows_per_w = x_hbm.shape[0] // NW
    wid = lax.axis_index("s") * NC + lax.axis_index("c")
    rbase = wid * rows_per_w

    def idx_start(slot, row):
        pltpu.make_async_copy(
            x_hbm.at[rbase + row], idx_v.at[slot], isem.at[slot]
        ).start()

    def idx_wait(slot):
        pltpu.make_async_copy(
            x_hbm.at[rbase], idx_v.at[slot], isem.at[slot]
        ).wait()

    def gather_start(b, slot):
        for off, n in SPLITS:
            pltpu.make_async_copy(
                table_hbm.at[idx_v.at[slot].at[pl.ds(off, n)]],
                rows.at[b].at[pl.ds(off, n)],
                gsem.at[b],
            ).start()

    def gather_wait(b):
        for off, n in SPLITS:
            pltpu.make_async_copy(
                table_hbm.at[idx_v.at[0].at[pl.ds(off, n)]],
                rows.at[b].at[pl.ds(off, n)],
                gsem.at[b],
            ).wait()

    def out_start(b, row):
        pltpu.make_async_copy(
            rows.at[b],
            out_hbm.at[pl.ds((rbase + row) * SEQ, SEQ)],
            osem.at[b],
        ).start()

    def out_wait(b):
        pltpu.make_async_copy(
            rows.at[b], out_hbm.at[pl.ds(0, SEQ)], osem.at[b]
        ).wait()

    for j in range(NBUF):
        idx_start(j, j)
    for j in range(LEAD):
        idx_wait(j)
        gather_start(j, j)

    for i in range(NBUF):
        b, s = i, (i + LEAD) % NBUF
        gather_wait(b)
        out_start(b, i)
        idx_start(b, i + NBUF)
        if i >= LEAD:
            out_wait(s)
        idx_wait(s)
        gather_start(s, s)

    @pl.loop(1, rows_per_w // NBUF)
    def _group(g):
        for b in range(NBUF):
            row = g * NBUF + b
            s = (b + LEAD) % NBUF
            gather_wait(b)
            out_start(b, row)

            @pl.when(row + NBUF < rows_per_w)
            def _():
                idx_start(b, row + NBUF)

            @pl.when(row + LEAD < rows_per_w)
            def _():
                out_wait(s)
                idx_wait(s)
                gather_start(s, s)

    for b in range(NBUF):
        out_wait(b)


def _sc_gather(table_f, x):
    n_tok = x.shape[0] * x.shape[1]
    mesh = plsc.VectorSubcoreMesh(core_axis_name="c", subcore_axis_name="s")
    run = pl.kernel(
        _gather_body,
        out_type=jax.ShapeDtypeStruct((n_tok, 2 * D_MODEL), jnp.float32),
        mesh=mesh,
        compiler_params=pltpu.CompilerParams(use_tc_tiling_on_sc=False),
        scratch_types=[
            pltpu.VMEM((NBUF, SEQ), jnp.int32),
            pltpu.VMEM((NBUF, SEQ, 2 * D_MODEL), jnp.float32),
            pltpu.SemaphoreType.DMA((NBUF,)),
            pltpu.SemaphoreType.DMA((NBUF,)),
            pltpu.SemaphoreType.DMA((NBUF,)),
        ],
    )
    return run(table_f, x)


def kernel(x, table):
    table_f = _format_table(table.T)
    out = _sc_gather(table_f, x)
    return out[:, :D_MODEL].reshape(x.shape[0], x.shape[1], D_MODEL)


# speedup vs baseline: 1.3134x; 1.0005x over previous
import functools

import jax
import jax.numpy as jnp
from jax import lax
from jax.experimental import pallas as pl
from jax.experimental.pallas import tpu as pltpu
from jax.experimental.pallas import tpu_sc as plsc

D_MODEL = 64
SCALE = 8.0
NC = 2
NS = 16
NW = NC * NS
SEQ = 200
SPLITS = ((0, 104), (104, 96))
NBUF = 4
LEAD = 3
FMT_BLOCK = 24576


def _fmt_body(t_ref, o_ref):
    t8 = (t_ref[...] * SCALE).T
    o_ref[...] = jnp.concatenate([t8, t8], axis=1)


def _format_table(table_t):
    vocab = table_t.shape[1]
    return pl.pallas_call(
        _fmt_body,
        grid=(pl.cdiv(vocab, FMT_BLOCK),),
        in_specs=[pl.BlockSpec((D_MODEL, FMT_BLOCK), lambda i: (0, i))],
        out_specs=pl.BlockSpec((FMT_BLOCK, 2 * D_MODEL), lambda i: (i, 0)),
        out_shape=jax.ShapeDtypeStruct((vocab, 2 * D_MODEL), jnp.float32),
        compiler_params=pltpu.CompilerParams(
            dimension_semantics=("arbitrary",)),
    )(table_t)


def _gather_body(table_hbm, x_hbm, out_hbm, idx_v, rows, gsem, osem, isem):
    rows_per_w = x_hbm.shape[0] // NW
    wid = lax.axis_index("s") * NC + lax.axis_index("c")
    rbase = wid * rows_per_w

    def idx_start(slot, row):
        pltpu.make_async_copy(
            x_hbm.at[rbase + row], idx_v.at[slot], isem.at[slot]
        ).start()

    def idx_wait(slot):
        pltpu.make_async_copy(
            x_hbm.at[rbase], idx_v.at[slot], isem.at[slot]
        ).wait()

    def gather_start(b, slot):
        for off, n in SPLITS:
            pltpu.make_async_copy(
                table_hbm.at[idx_v.at[slot].at[pl.ds(off, n)]],
                rows.at[b].at[pl.ds(off, n)],
                gsem.at[b],
            ).start()

    def gather_wait(b):
        for off, n in SPLITS:
            pltpu.make_async_copy(
                table_hbm.at[idx_v.at[0].at[pl.ds(off, n)]],
                rows.at[b].at[pl.ds(off, n)],
                gsem.at[b],
            ).wait()

    def out_start(b, row):
        pltpu.make_async_copy(
            rows.at[b],
            out_hbm.at[pl.ds((rbase + row) * SEQ, SEQ)],
            osem.at[b],
        ).start()

    def out_wait(b):
        pltpu.make_async_copy(
            rows.at[b], out_hbm.at[pl.ds(0, SEQ)], osem.at[b]
        ).wait()

    for j in range(NBUF):
        idx_start(j, j)
    for j in range(LEAD):
        idx_wait(j)
        gather_start(j, j)

    for i in range(NBUF):
        b, s = i, (i + LEAD) % NBUF
        gather_wait(b)
        out_start(b, i)
        idx_start(b, i + NBUF)
        if i >= LEAD:
            out_wait(s)
        idx_wait(s)
        gather_start(s, s)

    @pl.loop(1, rows_per_w // NBUF)
    def _group(g):
        for b in range(NBUF):
            row = g * NBUF + b
            s = (b + LEAD) % NBUF
            gather_wait(b)
            out_start(b, row)

            @pl.when(row + NBUF < rows_per_w)
            def _():
                idx_start(b, row + NBUF)

            @pl.when(row + LEAD < rows_per_w)
            def _():
                out_wait(s)
                idx_wait(s)
                gather_start(s, s)

    for b in range(NBUF):
        out_wait(b)


def _sc_gather(table_f, x):
    n_tok = x.shape[0] * x.shape[1]
    mesh = plsc.VectorSubcoreMesh(core_axis_name="c", subcore_axis_name="s")
    run = pl.kernel(
        _gather_body,
        out_type=jax.ShapeDtypeStruct((n_tok, 2 * D_MODEL), jnp.float32),
        mesh=mesh,
        compiler_params=pltpu.CompilerParams(use_tc_tiling_on_sc=False),
        scratch_types=[
            pltpu.VMEM((NBUF, SEQ), jnp.int32),
            pltpu.VMEM((NBUF, SEQ, 2 * D_MODEL), jnp.float32),
            pltpu.SemaphoreType.DMA((NBUF,)),
            pltpu.SemaphoreType.DMA((NBUF,)),
            pltpu.SemaphoreType.DMA((NBUF,)),
        ],
    )
    return run(table_f, x)


def kernel(x, table):
    table_f = _format_table(table.T)
    out = _sc_gather(table_f, x)
    return out[:, :D_MODEL].reshape(x.shape[0], x.shape[1], D_MODEL)
